# baseline (speedup 1.0000x reference)
import functools

import jax
import jax.numpy as jnp
from jax.experimental import pallas as pl
from jax.experimental.pallas import tpu as pltpu


def _scse_ring_kernel(x_hbm, w1_ref, b1_ref, w2_ref, b2_ref, ws_ref, bs_ref,
                      o_hbm, ibuf, obuf, in_sem, out_sem,
                      *, n_steps, pair, n_ch, kbuf, inv_hw):
    def start_in(n, slot):
        pltpu.make_async_copy(x_hbm.at[pl.ds(n, 1)], ibuf.at[pl.ds(slot, 1)],
                              in_sem.at[slot]).start()

    def wait_in(slot):
        pltpu.make_async_copy(x_hbm.at[pl.ds(0, 1)], ibuf.at[pl.ds(slot, 1)],
                              in_sem.at[slot]).wait()

    def start_out(n, slot):
        pltpu.make_async_copy(obuf.at[pl.ds(slot, 1)], o_hbm.at[pl.ds(n, 1)],
                              out_sem.at[slot]).start()

    def wait_out(slot):
        pltpu.make_async_copy(obuf.at[pl.ds(slot, 1)], o_hbm.at[pl.ds(0, 1)],
                              out_sem.at[slot]).wait()

    for k in range(kbuf):
        start_in(k, k)

    def body(n, _):
        slot = jax.lax.rem(n, kbuf)
        wait_in(slot)

        @pl.when(n >= kbuf)
        def _():
            wait_out(slot)

        xp = ibuf[pl.ds(slot, 1)][0]

        for i in range(pair):
            x = xp[i * n_ch:(i + 1) * n_ch]

            mean = jnp.sum(x, axis=1, keepdims=True) * inv_hw
            z = jnp.sum(mean * w1_ref[...], axis=0, keepdims=True)
            z = jnp.maximum(z + b1_ref[...], 0.0)
            s = jnp.sum(w2_ref[...] * z, axis=1, keepdims=True)
            s = jax.nn.sigmoid(s + b2_ref[...])

            q = jax.lax.dot_general(ws_ref[...], x, (((0,), (0,)), ((), ())),
                                    preferred_element_type=jnp.float32)
            q = jax.nn.sigmoid(q + bs_ref[0])

            obuf[0 if kbuf == 1 else slot, i * n_ch:(i + 1) * n_ch] = x * (s + q)
        start_out(n, slot)

        @pl.when(n + kbuf < n_steps)
        def _():
            start_in(n + kbuf, slot)

        return 0

    jax.lax.fori_loop(0, n_steps, body, 0, unroll=False)

    for p in range(max(n_steps - kbuf, 0), n_steps):
        wait_out(p % kbuf)


def kernel(x_nchw, w1, b1, w2, b2, ws, bs):
    N, C, H, W = x_nchw.shape
    HW = H * W
    dtype = x_nchw.dtype
    x = x_nchw.reshape(N, C, HW)

    HWp = ((HW + 127) // 128) * 128
    if HWp != HW:
        x = jnp.pad(x, ((0, 0), (0, 0), (0, HWp - HW)))

    pair = 2 if N % 2 == 0 else 1
    n_steps = N // pair
    kbuf = min(3, n_steps)
    xs = x.reshape(n_steps, pair * C, HWp)

    body = functools.partial(_scse_ring_kernel, n_steps=n_steps, pair=pair,
                             n_ch=C, kbuf=kbuf, inv_hw=1.0 / float(HW))
    out = pl.pallas_call(
        body,
        out_shape=jax.ShapeDtypeStruct((n_steps, pair * C, HWp), dtype),
        in_specs=[
            pl.BlockSpec(memory_space=pltpu.MemorySpace.HBM),
            pl.BlockSpec(memory_space=pltpu.MemorySpace.VMEM),
            pl.BlockSpec(memory_space=pltpu.MemorySpace.VMEM),
            pl.BlockSpec(memory_space=pltpu.MemorySpace.VMEM),
            pl.BlockSpec(memory_space=pltpu.MemorySpace.VMEM),
            pl.BlockSpec(memory_space=pltpu.MemorySpace.VMEM),
            pl.BlockSpec(memory_space=pltpu.MemorySpace.SMEM),
        ],
        out_specs=pl.BlockSpec(memory_space=pltpu.MemorySpace.HBM),
        scratch_shapes=[
            pltpu.VMEM((kbuf, pair * C, HWp), jnp.float32),
            pltpu.VMEM((kbuf, pair * C, HWp), jnp.float32),
            pltpu.SemaphoreType.DMA((kbuf,)),
            pltpu.SemaphoreType.DMA((kbuf,)),
        ],
        compiler_params=pltpu.CompilerParams(
            vmem_limit_bytes=60 * 1024 * 1024,
        ),
        cost_estimate=pl.CostEstimate(
            flops=6 * N * C * HWp,
            transcendentals=N * (HWp + C),
            bytes_accessed=2 * N * C * HWp * dtype.itemsize,
        ),
    )(xs, w1, b1, w2, b2, ws, bs)

    out = out.reshape(N, C, HWp)
    if HWp != HW:
        out = out[:, :, :HW]
    return out.reshape(N, C, H, W)


# speedup vs baseline: 2.2323x; 2.2323x over previous
import functools

import jax
import jax.numpy as jnp
from jax.experimental import pallas as pl
from jax.experimental.pallas import tpu as pltpu


def _scse_block_kernel(x_ref, w1_ref, b1_ref, w2_ref, b2_ref, ws_ref, bs_ref,
                       o_ref, *, nb, hw, tw, inv_hw):
    ws = ws_ref[...]
    for i in range(nb):
        x = x_ref[i]

        mean = jnp.sum(x, axis=1, keepdims=True) * inv_hw
        z = jnp.sum(mean * w1_ref[...], axis=0, keepdims=True)
        z = jnp.maximum(z + b1_ref[...], 0.0)
        s = jnp.sum(w2_ref[...] * z, axis=1, keepdims=True)
        s = jax.nn.sigmoid(s + b2_ref[...])

        q = jax.lax.dot_general(ws, x, (((0,), (0,)), ((), ())),
                                preferred_element_type=jnp.float32)
        q = jax.nn.sigmoid(q + bs_ref[0])

        for t in range(0, hw, tw):
            o_ref[i, :, t:t + tw] = x[:, t:t + tw] * (s + q[:, t:t + tw])


def kernel(x_nchw, w1, b1, w2, b2, ws, bs):
    N, C, H, W = x_nchw.shape
    HW = H * W
    dtype = x_nchw.dtype
    x = x_nchw.reshape(N, C, HW)

    HWp = ((HW + 127) // 128) * 128
    if HWp != HW:
        x = jnp.pad(x, ((0, 0), (0, 0), (0, HWp - HW)))

    plane_bytes = C * HWp * dtype.itemsize
    nb = 1
    for cand in (4, 2, 1):
        if N % cand == 0 and 4 * cand * plane_bytes <= 44 * 1024 * 1024:
            nb = cand
            break

    tw = 128

    body = functools.partial(_scse_block_kernel, nb=nb, hw=HWp, tw=tw,
                             inv_hw=1.0 / float(HW))
    out = pl.pallas_call(
        body,
        out_shape=jax.ShapeDtypeStruct((N, C, HWp), dtype),
        grid=(N // nb,),
        in_specs=[
            pl.BlockSpec((nb, C, HWp), lambda n: (n, 0, 0)),
            pl.BlockSpec(w1.shape, lambda n: (0, 0)),
            pl.BlockSpec(b1.shape, lambda n: (0, 0)),
            pl.BlockSpec(w2.shape, lambda n: (0, 0)),
            pl.BlockSpec(b2.shape, lambda n: (0, 0)),
            pl.BlockSpec(ws.shape, lambda n: (0, 0)),
            pl.BlockSpec(memory_space=pltpu.MemorySpace.SMEM),
        ],
        out_specs=pl.BlockSpec((nb, C, HWp), lambda n: (n, 0, 0)),
        compiler_params=pltpu.CompilerParams(
            dimension_semantics=("parallel",),
            vmem_limit_bytes=52 * 1024 * 1024,
        ),
        cost_estimate=pl.CostEstimate(
            flops=6 * N * C * HWp,
            transcendentals=N * (HWp + C),
            bytes_accessed=2 * N * C * HWp * dtype.itemsize,
        ),
    )(x, w1, b1, w2, b2, ws, bs)

    if HWp != HW:
        out = out[:, :, :HW]
    return out.reshape(N, C, H, W)
